# Initial kernel scaffold; baseline (speedup 1.0000x reference)
import jax
import jax.numpy as jnp
from jax.experimental import pallas as pl


def kernel(edge_index, W_weight, W_bias):
    raise NotImplementedError("write your pallas kernel here")


# speedup vs baseline: 3.3674x; 3.3674x over previous
import functools

import jax
import jax.numpy as jnp
from jax import lax
from jax.experimental import pallas as pl
from jax.experimental.pallas import tpu as pltpu
from jax.experimental.pallas import tpu_sc as plsc

N = 10000
C = 128
E = 320000

H = C // 2
NC = 2
NS = 16
L = 16

EPT = E // NS
CH = 80
NSTEPS = EPT // CH
BR = 80
NBLK = N // BR


def _sc_body(row_hbm, col_hbm, wt0_hbm, wt1_hbm, bias_hbm, out_hbm,
             colidx_v, rowidx_v, rows_v, stage_v, bias_v, acc_sh, sem):
    c = lax.axis_index("c")
    s = lax.axis_index("s")

    @pl.loop(0, BR)
    def _(i):
        for kk in range(H // L):
            stage_v[i, pl.ds(kk * L, L)] = jnp.zeros((L,), jnp.float32)

    @pl.loop(s, NBLK, step=NS)
    def _(b):
        pltpu.sync_copy(stage_v, acc_sh.at[pl.ds(b * BR, BR)])

    pltpu.sync_copy(bias_hbm.at[pl.ds(c * H, H)], bias_v)
    plsc.subcore_barrier()

    def run_edges(wt_ref):
        @pl.loop(0, NSTEPS)
        def _(j):
            base = s * EPT + j * CH
            pltpu.sync_copy(row_hbm.at[pl.ds(base, CH)], rowidx_v)
            pltpu.sync_copy(col_hbm.at[pl.ds(base, CH)], colidx_v)
            pltpu.async_copy(wt_ref.at[colidx_v], rows_v, sem).wait()
            pltpu.sync_copy(rows_v, acc_sh.at[rowidx_v], add=True)

    @pl.when(c == 0)
    def _():
        run_edges(wt0_hbm)

    @pl.when(c == 1)
    def _():
        run_edges(wt1_hbm)

    plsc.subcore_barrier()

    @pl.loop(s, NBLK, step=NS)
    def _(b):
        r0 = b * BR
        pltpu.sync_copy(acc_sh.at[pl.ds(r0, BR)], stage_v)

        @pl.loop(0, BR)
        def _(i):
            for kk in range(H // L):
                sl = pl.ds(kk * L, L)
                stage_v[i, sl] = stage_v[i, sl] + bias_v[sl]

        pltpu.sync_copy(stage_v, out_hbm.at[c, pl.ds(r0, BR)])


_sc_link = functools.partial(
    pl.kernel,
    out_type=jax.ShapeDtypeStruct((NC, N, H), jnp.float32),
    mesh=plsc.VectorSubcoreMesh(core_axis_name="c", subcore_axis_name="s"),
    compiler_params=pltpu.CompilerParams(use_tc_tiling_on_sc=False),
    scratch_types=[
        pltpu.VMEM((CH,), jnp.int32),
        pltpu.VMEM((CH,), jnp.int32),
        pltpu.VMEM((CH, H), jnp.float32),
        pltpu.VMEM((BR, H), jnp.float32),
        pltpu.VMEM((H,), jnp.float32),
        pltpu.VMEM_SHARED((N, H), jnp.float32),
        pltpu.SemaphoreType.DMA,
    ],
)(_sc_body)


def kernel(edge_index, W_weight, W_bias):
    row = edge_index[0]
    col = edge_index[1]
    row = row - jnp.min(row)
    wt = W_weight.T
    halves = _sc_link(row, col, wt[:, :H], wt[:, H:], W_bias)
    return jnp.concatenate([halves[0], halves[1]], axis=-1)


# speedup vs baseline: 7.8011x; 2.3167x over previous
import functools

import jax
import jax.numpy as jnp
from jax import lax
from jax.experimental import pallas as pl
from jax.experimental.pallas import tpu as pltpu
from jax.experimental.pallas import tpu_sc as plsc

N = 10000
C = 128
E = 320000

H = C // 2
NC = 2
NS = 16
L = 16

EPT = E // NS
CH = 125
K = 8
NSB = EPT // (K * CH)
BR = 80
NBLK = N // BR


def _sc_body(row_hbm, col_hbm, wt0_hbm, wt1_hbm, bias_hbm, out_hbm,
             colidx_v, rowidx_v, rows_v, stage_v, bias_v, acc_sh,
             gsems, ssem):
    c = lax.axis_index("c")
    s = lax.axis_index("s")

    @pl.loop(0, BR)
    def _(i):
        for kk in range(H // L):
            stage_v[i, pl.ds(kk * L, L)] = jnp.zeros((L,), jnp.float32)

    @pl.loop(s, NBLK, step=NS)
    def _(b):
        pltpu.sync_copy(stage_v, acc_sh.at[pl.ds(b * BR, BR)])

    pltpu.sync_copy(bias_hbm.at[pl.ds(c * H, H)], bias_v)
    plsc.subcore_barrier()

    def run_edges(wt_ref):
        @pl.loop(0, NSB)
        def _(b):
            pltpu.sync_copy(row_hbm.at[s, b], rowidx_v)
            pltpu.sync_copy(col_hbm.at[s, b], colidx_v)
            gds = [
                pltpu.async_copy(wt_ref.at[colidx_v.at[k]], rows_v.at[k],
                                 gsems.at[k])
                for k in range(K)
            ]
            sds = []
            for k in range(K):
                gds[k].wait()
                sds.append(
                    pltpu.async_copy(rows_v.at[k], acc_sh.at[rowidx_v.at[k]],
                                     ssem, add=True))
            for d in sds:
                d.wait()

    @pl.when(c == 0)
    def _():
        run_edges(wt0_hbm)

    @pl.when(c == 1)
    def _():
        run_edges(wt1_hbm)

    plsc.subcore_barrier()

    @pl.loop(s, NBLK, step=NS)
    def _(b):
        r0 = b * BR
        pltpu.sync_copy(acc_sh.at[pl.ds(r0, BR)], stage_v)

        @pl.loop(0, BR)
        def _(i):
            for kk in range(H // L):
                sl = pl.ds(kk * L, L)
                stage_v[i, sl] = stage_v[i, sl] + bias_v[sl]

        pltpu.sync_copy(stage_v, out_hbm.at[c, pl.ds(r0, BR)])


_sc_link = functools.partial(
    pl.kernel,
    out_type=jax.ShapeDtypeStruct((NC, N, H), jnp.float32),
    mesh=plsc.VectorSubcoreMesh(core_axis_name="c", subcore_axis_name="s"),
    compiler_params=pltpu.CompilerParams(use_tc_tiling_on_sc=False),
    scratch_types=[
        pltpu.VMEM((K, CH), jnp.int32),
        pltpu.VMEM((K, CH), jnp.int32),
        pltpu.VMEM((K, CH, H), jnp.float32),
        pltpu.VMEM((BR, H), jnp.float32),
        pltpu.VMEM((H,), jnp.float32),
        pltpu.VMEM_SHARED((N, H), jnp.float32),
        pltpu.SemaphoreType.DMA((K,)),
        pltpu.SemaphoreType.DMA,
    ],
)(_sc_body)


def kernel(edge_index, W_weight, W_bias):
    row = edge_index[0]
    col = edge_index[1]
    row = row - jnp.min(row)
    wt = W_weight.T
    row4 = row.reshape(NS, NSB, K, CH)
    col4 = col.reshape(NS, NSB, K, CH)
    halves = _sc_link(row4, col4, wt[:, :H], wt[:, H:], W_bias)
    return jnp.concatenate([halves[0], halves[1]], axis=-1)


# speedup vs baseline: 10.3586x; 1.3278x over previous
import functools

import jax
import jax.numpy as jnp
from jax import lax
from jax.experimental import pallas as pl
from jax.experimental.pallas import tpu as pltpu
from jax.experimental.pallas import tpu_sc as plsc

N = 10000
C = 128
E = 320000

H = C // 2
NC = 2
NS = 16
L = 16

EPT = E // NS
CH = 125
K = 8
NSB = EPT // (K * CH)
NO = NSB // 2
BR = 80
NBLK = N // BR


def _sc_body(row_hbm, col_hbm, wt0_hbm, wt1_hbm, bias_hbm, out_hbm,
             colidx_v, rowidx_v, rows_v, stage_v, bias_v, acc_sh,
             gsems, ssem, isems):
    c = lax.axis_index("c")
    s = lax.axis_index("s")

    @pl.loop(0, BR)
    def _(i):
        for kk in range(H // L):
            stage_v[i, pl.ds(kk * L, L)] = jnp.zeros((L,), jnp.float32)

    @pl.loop(s, NBLK, step=NS)
    def _(b):
        pltpu.sync_copy(stage_v, acc_sh.at[pl.ds(b * BR, BR)])

    pltpu.sync_copy(bias_hbm.at[pl.ds(c * H, H)], bias_v)
    plsc.subcore_barrier()

    def run_edges(wt_ref):
        i0 = pltpu.async_copy(row_hbm.at[s, 0, 0], rowidx_v.at[0],
                              isems.at[0])
        i1 = pltpu.async_copy(col_hbm.at[s, 0, 0], colidx_v.at[0],
                              isems.at[0])
        del i0, i1

        @pl.loop(0, NO)
        def _(o):
            for nb in range(2):
                bb_gt0 = (o > 0) if nb == 0 else True

                def drain():
                    for k in range(K):
                        pltpu.make_async_copy(
                            wt_ref.at[pl.ds(0, CH)], rows_v.at[k],
                            ssem).wait()

                if bb_gt0 is True:
                    drain()
                else:
                    pl.when(bb_gt0)(drain)

                def prefetch(oo, nnb):
                    pltpu.async_copy(row_hbm.at[s, oo, nnb],
                                     rowidx_v.at[nnb], isems.at[nnb])
                    pltpu.async_copy(col_hbm.at[s, oo, nnb],
                                     colidx_v.at[nnb], isems.at[nnb])

                if nb == 0:
                    prefetch(o, 1)
                else:
                    pl.when(o < NO - 1)(lambda: prefetch(o + 1, 0))

                pltpu.make_async_copy(row_hbm.at[s, 0, 0],
                                      rowidx_v.at[nb], isems.at[nb]).wait()
                pltpu.make_async_copy(col_hbm.at[s, 0, 0],
                                      colidx_v.at[nb], isems.at[nb]).wait()

                gds = [
                    pltpu.async_copy(wt_ref.at[colidx_v.at[nb, k]],
                                     rows_v.at[k], gsems.at[k])
                    for k in range(K)
                ]
                for k in range(K):
                    gds[k].wait()
                    pltpu.async_copy(rows_v.at[k],
                                     acc_sh.at[rowidx_v.at[nb, k]],
                                     ssem, add=True)

        for k in range(K):
            pltpu.make_async_copy(wt_ref.at[pl.ds(0, CH)], rows_v.at[k],
                                  ssem).wait()

    @pl.when(c == 0)
    def _():
        run_edges(wt0_hbm)

    @pl.when(c == 1)
    def _():
        run_edges(wt1_hbm)

    plsc.subcore_barrier()

    @pl.loop(s, NBLK, step=NS)
    def _(b):
        r0 = b * BR
        pltpu.sync_copy(acc_sh.at[pl.ds(r0, BR)], stage_v)

        @pl.loop(0, BR)
        def _(i):
            for kk in range(H // L):
                sl = pl.ds(kk * L, L)
                stage_v[i, sl] = stage_v[i, sl] + bias_v[sl]

        pltpu.sync_copy(stage_v,
                        out_hbm.at[pl.ds(r0, BR), pl.ds(c * H, H)])


_sc_link = functools.partial(
    pl.kernel,
    out_type=jax.ShapeDtypeStruct((N, C), jnp.float32),
    mesh=plsc.VectorSubcoreMesh(core_axis_name="c", subcore_axis_name="s"),
    compiler_params=pltpu.CompilerParams(use_tc_tiling_on_sc=False),
    scratch_types=[
        pltpu.VMEM((2, K, CH), jnp.int32),
        pltpu.VMEM((2, K, CH), jnp.int32),
        pltpu.VMEM((K, CH, H), jnp.float32),
        pltpu.VMEM((BR, H), jnp.float32),
        pltpu.VMEM((H,), jnp.float32),
        pltpu.VMEM_SHARED((N, H), jnp.float32),
        pltpu.SemaphoreType.DMA((K,)),
        pltpu.SemaphoreType.DMA,
        pltpu.SemaphoreType.DMA((2,)),
    ],
)(_sc_body)


def kernel(edge_index, W_weight, W_bias):
    row = edge_index[0]
    col = edge_index[1]
    row = row - jnp.min(row)
    row5 = row.reshape(NS, NO, 2, K, CH)
    col5 = col.reshape(NS, NO, 2, K, CH)
    wt0 = W_weight[:H].T
    wt1 = W_weight[H:].T
    return _sc_link(row5, col5, wt0, wt1, W_bias)


# speedup vs baseline: 11.0260x; 1.0644x over previous
import functools

import jax
import jax.numpy as jnp
from jax import lax
from jax.experimental import pallas as pl
from jax.experimental.pallas import tpu as pltpu
from jax.experimental.pallas import tpu_sc as plsc

N = 10000
C = 128
E = 320000

H = C // 2
NC = 2
NS = 16
L = 16

EPT = E // NS
CH = 125
K = 8
NSB = EPT // (K * CH)
NO = NSB // 2
BR = 80
NBLK = N // BR


def _sc_body(row_hbm, col_hbm, wt0_hbm, wt1_hbm, bias_hbm, out_hbm,
             colidx_v, rowidx_v, rows_v, stage_v, bias_v, acc_sh,
             gsems, ssem, isems):
    c = lax.axis_index("c")
    s = lax.axis_index("s")

    pltpu.sync_copy(bias_hbm.at[pl.ds(c * H, H)], bias_v)

    @pl.loop(0, BR)
    def _(i):
        for kk in range(H // L):
            sl = pl.ds(kk * L, L)
            stage_v[i, sl] = bias_v[sl]

    @pl.loop(s, NBLK, step=NS)
    def _(b):
        pltpu.sync_copy(stage_v, acc_sh.at[pl.ds(b * BR, BR)])

    plsc.subcore_barrier()

    def run_edges(wt_ref):
        i0 = pltpu.async_copy(row_hbm.at[s, 0, 0], rowidx_v.at[0],
                              isems.at[0])
        i1 = pltpu.async_copy(col_hbm.at[s, 0, 0], colidx_v.at[0],
                              isems.at[0])
        del i0, i1

        @pl.loop(0, NO)
        def _(o):
            for nb in range(2):
                bb_gt0 = (o > 0) if nb == 0 else True

                def drain():
                    for k in range(K):
                        pltpu.make_async_copy(
                            wt_ref.at[pl.ds(0, CH)], rows_v.at[k],
                            ssem).wait()

                if bb_gt0 is True:
                    drain()
                else:
                    pl.when(bb_gt0)(drain)

                def prefetch(oo, nnb):
                    pltpu.async_copy(row_hbm.at[s, oo, nnb],
                                     rowidx_v.at[nnb], isems.at[nnb])
                    pltpu.async_copy(col_hbm.at[s, oo, nnb],
                                     colidx_v.at[nnb], isems.at[nnb])

                if nb == 0:
                    prefetch(o, 1)
                else:
                    pl.when(o < NO - 1)(lambda: prefetch(o + 1, 0))

                pltpu.make_async_copy(row_hbm.at[s, 0, 0],
                                      rowidx_v.at[nb], isems.at[nb]).wait()
                pltpu.make_async_copy(col_hbm.at[s, 0, 0],
                                      colidx_v.at[nb], isems.at[nb]).wait()

                gds = [
                    pltpu.async_copy(wt_ref.at[colidx_v.at[nb, k]],
                                     rows_v.at[k], gsems.at[k])
                    for k in range(K)
                ]
                for k in range(K):
                    gds[k].wait()
                    pltpu.async_copy(rows_v.at[k],
                                     acc_sh.at[rowidx_v.at[nb, k]],
                                     ssem, add=True)

        for k in range(K):
            pltpu.make_async_copy(wt_ref.at[pl.ds(0, CH)], rows_v.at[k],
                                  ssem).wait()

    @pl.when(c == 0)
    def _():
        run_edges(wt0_hbm)

    @pl.when(c == 1)
    def _():
        run_edges(wt1_hbm)

    plsc.subcore_barrier()

    @pl.loop(s, NBLK, step=NS)
    def _(b):
        r0 = b * BR
        pltpu.sync_copy(acc_sh.at[pl.ds(r0, BR)], stage_v)
        pltpu.sync_copy(stage_v,
                        out_hbm.at[pl.ds(r0, BR), pl.ds(c * H, H)])


_sc_link = functools.partial(
    pl.kernel,
    out_type=jax.ShapeDtypeStruct((N, C), jnp.float32),
    mesh=plsc.VectorSubcoreMesh(core_axis_name="c", subcore_axis_name="s"),
    compiler_params=pltpu.CompilerParams(use_tc_tiling_on_sc=False),
    scratch_types=[
        pltpu.VMEM((2, K, CH), jnp.int32),
        pltpu.VMEM((2, K, CH), jnp.int32),
        pltpu.VMEM((K, CH, H), jnp.float32),
        pltpu.VMEM((BR, H), jnp.float32),
        pltpu.VMEM((H,), jnp.float32),
        pltpu.VMEM_SHARED((N, H), jnp.float32),
        pltpu.SemaphoreType.DMA((K,)),
        pltpu.SemaphoreType.DMA,
        pltpu.SemaphoreType.DMA((2,)),
    ],
)(_sc_body)


def kernel(edge_index, W_weight, W_bias):
    row = edge_index[0]
    col = edge_index[1]
    row = row - jnp.min(row)
    row5 = row.reshape(NS, NO, 2, K, CH)
    col5 = col.reshape(NS, NO, 2, K, CH)
    wt0 = W_weight[:H].T
    wt1 = W_weight[H:].T
    return _sc_link(row5, col5, wt0, wt1, W_bias)


# speedup vs baseline: 11.4835x; 1.0415x over previous
import functools

import jax
import jax.numpy as jnp
from jax import lax
from jax.experimental import pallas as pl
from jax.experimental.pallas import tpu as pltpu
from jax.experimental.pallas import tpu_sc as plsc

N = 10000
C = 128
E = 320000

H = C // 2
NC = 2
NS = 16
L = 16

EPT = E // NS
CH = 125
K = 10
NSB = EPT // (K * CH)
NO = NSB // 2
BR = 80
NBLK = N // BR


def _sc_body(row_hbm, col_hbm, wt0_hbm, wt1_hbm, bias_hbm, out_hbm,
             colidx_v, rowidx_v, rows_v, stage_v, bias_v, acc_sh,
             gsems, ssem, isems):
    c = lax.axis_index("c")
    s = lax.axis_index("s")

    pltpu.sync_copy(bias_hbm.at[pl.ds(c * H, H)], bias_v)

    @pl.loop(0, BR)
    def _(i):
        for kk in range(H // L):
            sl = pl.ds(kk * L, L)
            stage_v[i, sl] = bias_v[sl]

    @pl.loop(s, NBLK, step=NS)
    def _(b):
        pltpu.sync_copy(stage_v, acc_sh.at[pl.ds(b * BR, BR)])

    plsc.subcore_barrier()

    def run_edges(wt_ref):
        i0 = pltpu.async_copy(row_hbm.at[s, 0, 0], rowidx_v.at[0],
                              isems.at[0])
        i1 = pltpu.async_copy(col_hbm.at[s, 0, 0], colidx_v.at[0],
                              isems.at[0])
        del i0, i1

        @pl.loop(0, NO)
        def _(o):
            for nb in range(2):
                bb_gt0 = (o > 0) if nb == 0 else True

                def drain():
                    for k in range(K):
                        pltpu.make_async_copy(
                            wt_ref.at[pl.ds(0, CH)], rows_v.at[k],
                            ssem).wait()

                if bb_gt0 is True:
                    drain()
                else:
                    pl.when(bb_gt0)(drain)

                def prefetch(oo, nnb):
                    pltpu.async_copy(row_hbm.at[s, oo, nnb],
                                     rowidx_v.at[nnb], isems.at[nnb])
                    pltpu.async_copy(col_hbm.at[s, oo, nnb],
                                     colidx_v.at[nnb], isems.at[nnb])

                if nb == 0:
                    prefetch(o, 1)
                else:
                    pl.when(o < NO - 1)(lambda: prefetch(o + 1, 0))

                pltpu.make_async_copy(row_hbm.at[s, 0, 0],
                                      rowidx_v.at[nb], isems.at[nb]).wait()
                pltpu.make_async_copy(col_hbm.at[s, 0, 0],
                                      colidx_v.at[nb], isems.at[nb]).wait()

                gds = [
                    pltpu.async_copy(wt_ref.at[colidx_v.at[nb, k]],
                                     rows_v.at[k], gsems.at[k])
                    for k in range(K)
                ]
                for k in range(K):
                    gds[k].wait()
                    pltpu.async_copy(rows_v.at[k],
                                     acc_sh.at[rowidx_v.at[nb, k]],
                                     ssem, add=True)

        for k in range(K):
            pltpu.make_async_copy(wt_ref.at[pl.ds(0, CH)], rows_v.at[k],
                                  ssem).wait()

    @pl.when(c == 0)
    def _():
        run_edges(wt0_hbm)

    @pl.when(c == 1)
    def _():
        run_edges(wt1_hbm)

    plsc.subcore_barrier()

    @pl.loop(s, NBLK, step=NS)
    def _(b):
        r0 = b * BR
        pltpu.sync_copy(acc_sh.at[pl.ds(r0, BR)], stage_v)
        pltpu.sync_copy(stage_v,
                        out_hbm.at[pl.ds(r0, BR), pl.ds(c * H, H)])


_sc_link = functools.partial(
    pl.kernel,
    out_type=jax.ShapeDtypeStruct((N, C), jnp.float32),
    mesh=plsc.VectorSubcoreMesh(core_axis_name="c", subcore_axis_name="s"),
    compiler_params=pltpu.CompilerParams(use_tc_tiling_on_sc=False),
    scratch_types=[
        pltpu.VMEM((2, K, CH), jnp.int32),
        pltpu.VMEM((2, K, CH), jnp.int32),
        pltpu.VMEM((K, CH, H), jnp.float32),
        pltpu.VMEM((BR, H), jnp.float32),
        pltpu.VMEM((H,), jnp.float32),
        pltpu.VMEM_SHARED((N, H), jnp.float32),
        pltpu.SemaphoreType.DMA((K,)),
        pltpu.SemaphoreType.DMA,
        pltpu.SemaphoreType.DMA((2,)),
    ],
)(_sc_body)


def kernel(edge_index, W_weight, W_bias):
    row = edge_index[0]
    col = edge_index[1]
    row = row - jnp.min(row)
    row5 = row.reshape(NS, NO, 2, K, CH)
    col5 = col.reshape(NS, NO, 2, K, CH)
    wt0 = W_weight[:H].T
    wt1 = W_weight[H:].T
    return _sc_link(row5, col5, wt0, wt1, W_bias)


# speedup vs baseline: 12.3141x; 1.0723x over previous
import functools

import jax
import jax.numpy as jnp
from jax import lax
from jax.experimental import pallas as pl
from jax.experimental.pallas import tpu as pltpu
from jax.experimental.pallas import tpu_sc as plsc

N = 10000
C = 128
E = 320000

H = C // 2
NC = 2
NS = 16
L = 16

EPT = E // NS
CH = 125
K = 8
NSB = EPT // (K * CH)
NB = 4
NO = NSB // NB
BR = 80
NBLK = N // BR


def _sc_body(row_hbm, col_hbm, wt0_hbm, wt1_hbm, bias_hbm, out_hbm,
             colidx_v, rowidx_v, rows_v, stage_v, bias_v, acc_sh,
             gsems, ssems, isems):
    c = lax.axis_index("c")
    s = lax.axis_index("s")

    pltpu.async_copy(row_hbm.at[s, 0, 0], rowidx_v.at[0], isems.at[0])
    pltpu.async_copy(col_hbm.at[s, 0, 0], colidx_v.at[0], isems.at[0])

    pltpu.sync_copy(bias_hbm.at[pl.ds(c * H, H)], bias_v)

    @pl.loop(0, BR)
    def _(i):
        for kk in range(H // L):
            sl = pl.ds(kk * L, L)
            stage_v[i, sl] = bias_v[sl]

    @pl.loop(s, NBLK, step=NS)
    def _(b):
        pltpu.sync_copy(stage_v, acc_sh.at[pl.ds(b * BR, BR)])

    plsc.subcore_barrier()

    def run_edges(wt_ref):
        @pl.loop(0, NO)
        def _(o):
            for nb in range(NB):
                def prefetch(oo, nnb):
                    pltpu.async_copy(row_hbm.at[s, oo, nnb],
                                     rowidx_v.at[nnb], isems.at[nnb])
                    pltpu.async_copy(col_hbm.at[s, oo, nnb],
                                     colidx_v.at[nnb], isems.at[nnb])

                if nb < NB - 1:
                    prefetch(o, nb + 1)
                else:
                    pl.when(o < NO - 1)(lambda: prefetch(o + 1, 0))

                pltpu.make_async_copy(row_hbm.at[s, 0, 0],
                                      rowidx_v.at[nb], isems.at[nb]).wait()
                pltpu.make_async_copy(col_hbm.at[s, 0, 0],
                                      colidx_v.at[nb], isems.at[nb]).wait()

                bb_gt0 = True if nb > 0 else (o > 0)
                gds = []
                for k in range(K):
                    def drain_k(k=k):
                        pltpu.make_async_copy(
                            wt_ref.at[pl.ds(0, CH)], rows_v.at[k],
                            ssems.at[k]).wait()

                    if bb_gt0 is True:
                        drain_k()
                    else:
                        pl.when(bb_gt0)(drain_k)
                    gds.append(
                        pltpu.async_copy(wt_ref.at[colidx_v.at[nb, k]],
                                         rows_v.at[k], gsems.at[k]))
                for k in range(K):
                    gds[k].wait()
                    pltpu.async_copy(rows_v.at[k],
                                     acc_sh.at[rowidx_v.at[nb, k]],
                                     ssems.at[k], add=True)

        for k in range(K):
            pltpu.make_async_copy(wt_ref.at[pl.ds(0, CH)], rows_v.at[k],
                                  ssems.at[k]).wait()

    @pl.when(c == 0)
    def _():
        run_edges(wt0_hbm)

    @pl.when(c == 1)
    def _():
        run_edges(wt1_hbm)

    plsc.subcore_barrier()

    @pl.loop(s, NBLK, step=NS)
    def _(b):
        r0 = b * BR
        pltpu.sync_copy(acc_sh.at[pl.ds(r0, BR)], stage_v)
        pltpu.sync_copy(stage_v,
                        out_hbm.at[pl.ds(r0, BR), pl.ds(c * H, H)])


_sc_link = functools.partial(
    pl.kernel,
    out_type=jax.ShapeDtypeStruct((N, C), jnp.float32),
    mesh=plsc.VectorSubcoreMesh(core_axis_name="c", subcore_axis_name="s"),
    compiler_params=pltpu.CompilerParams(use_tc_tiling_on_sc=False),
    scratch_types=[
        pltpu.VMEM((NB, K, CH), jnp.int32),
        pltpu.VMEM((NB, K, CH), jnp.int32),
        pltpu.VMEM((K, CH, H), jnp.float32),
        pltpu.VMEM((BR, H), jnp.float32),
        pltpu.VMEM((H,), jnp.float32),
        pltpu.VMEM_SHARED((N, H), jnp.float32),
        pltpu.SemaphoreType.DMA((K,)),
        pltpu.SemaphoreType.DMA((K,)),
        pltpu.SemaphoreType.DMA((NB,)),
    ],
)(_sc_body)


def kernel(edge_index, W_weight, W_bias):
    row = edge_index[0]
    col = edge_index[1]
    row = row - jnp.min(row)
    row5 = row.reshape(NS, NO, NB, K, CH)
    col5 = col.reshape(NS, NO, NB, K, CH)
    wt0 = W_weight[:H].T
    wt1 = W_weight[H:].T
    return _sc_link(row5, col5, wt0, wt1, W_bias)
